# baseline (speedup 1.0000x reference)
import functools

import jax
import jax.numpy as jnp
from jax import lax
from jax.experimental import pallas as pl
from jax.experimental.pallas import tpu as pltpu
from jax.experimental.pallas import tpu_sc as plsc

N = 10000
D = 256
HALF = 128
E = 160000

NC = 2
NS = 16
CHUNK = 112
CHUNKS = 90
EPT = CHUNKS * CHUNK
E_PAD = NS * EPT
N_ACC = 10112
ZROWS = N_ACC // NS

_mesh = plsc.VectorSubcoreMesh(core_axis_name="c", subcore_axis_name="s")


@functools.partial(
    pl.kernel,
    out_type=jax.ShapeDtypeStruct((N_ACC,), jnp.float32),
    mesh=_mesh,
    scratch_types=[
        pltpu.VMEM((CHUNKS, 2, CHUNK), jnp.int32),
        pltpu.VMEM((CHUNK,), jnp.float32),
        pltpu.VMEM_SHARED((N_ACC,), jnp.float32),
    ],
)
def _deg_kernel(rc_hbm, zdeg_hbm, deg_out, col_v, ones_v, deg_sh):
    c = lax.axis_index("c")
    s = lax.axis_index("s")
    pltpu.sync_copy(rc_hbm.at[s], col_v)
    for k in range(CHUNK // 16):
        ones_v[pl.ds(k * 16, 16)] = jnp.ones((16,), jnp.float32)

    @pl.when(s == 0)
    def _():
        pltpu.sync_copy(zdeg_hbm, deg_sh)

    plsc.subcore_barrier()

    def body(j, carry):
        pltpu.sync_copy(ones_v, deg_sh.at[col_v.at[j, 1]], add=True)
        return carry

    lax.fori_loop(0, CHUNKS, body, 0)
    plsc.subcore_barrier()

    @pl.when((s == 0) & (c == 0))
    def _():
        pltpu.sync_copy(deg_sh, deg_out)


@functools.partial(
    pl.kernel,
    out_type=(
        jax.ShapeDtypeStruct((N_ACC, HALF), jnp.float32),
        jax.ShapeDtypeStruct((N_ACC, HALF), jnp.float32),
    ),
    mesh=_mesh,
    scratch_types=[
        pltpu.VMEM((6, 2, CHUNK), jnp.int32),
        pltpu.VMEM((3, CHUNK, HALF), jnp.float32),
        pltpu.VMEM_SHARED((N_ACC, HALF), jnp.float32),
        [pltpu.SemaphoreType.DMA] * 6,
        [pltpu.SemaphoreType.DMA] * 3,
        [pltpu.SemaphoreType.DMA] * 3,
    ],
)
def _mp_kernel(rc_hbm, xs_lo, xs_hi, zacc_hbm, out_lo, out_hi,
               rc_v, buf_v, acc_sh, sem_i, sem_g, sem_s):
    c = lax.axis_index("c")
    s = lax.axis_index("s")

    def run(xs_hbm, out_hbm):
        pltpu.sync_copy(zacc_hbm, acc_sh.at[pl.ds(s * ZROWS, ZROWS)])
        plsc.subcore_barrier()

        def i_copy(j, js):
            return pltpu.make_async_copy(
                rc_hbm.at[s, j], rc_v.at[js], sem_i[js])

        def g_copy(js, db):
            return pltpu.make_async_copy(
                xs_hbm.at[rc_v.at[js, 0]], buf_v.at[db], sem_g[db])

        def s_copy(js, db):
            return pltpu.make_async_copy(
                buf_v.at[db], acc_sh.at[rc_v.at[js, 1]], sem_s[db])

        for k in range(6):
            i_copy(k, k).start()
        i_copy(0, 0).wait()
        g_copy(0, 0).start()

        H = CHUNKS // 6

        def body(h, carry):
            j0 = 6 * h
            for k in range(6):
                j = j0 + k
                js = k
                ps = (k + 4) % 6
                db = k % 3
                pb = (k + 1) % 3
                g_copy(js, db).wait()
                s_copy(js, db).start(add=True)

                def retire_and_next(refill, nxt):
                    s_copy(ps, pb).wait()
                    if refill:
                        i_copy(j + 4, ps).start()
                    if nxt:
                        i_copy(j + 1, (k + 1) % 6).wait()
                        g_copy((k + 1) % 6, pb).start()

                if k < 2:
                    @pl.when(h > 0)
                    def _():
                        retire_and_next(True, False)
                    i_copy(j + 1, (k + 1) % 6).wait()
                    g_copy((k + 1) % 6, pb).start()
                elif k < 5:
                    s_copy(ps, pb).wait()

                    @pl.when(h < H - 1)
                    def _():
                        i_copy(j + 4, ps).start()
                    i_copy(j + 1, (k + 1) % 6).wait()
                    g_copy((k + 1) % 6, pb).start()
                else:
                    s_copy(ps, pb).wait()

                    @pl.when(h < H - 1)
                    def _():
                        i_copy(j + 4, ps).start()
                        i_copy(j + 1, (k + 1) % 6).wait()
                        g_copy((k + 1) % 6, pb).start()

            return carry

        lax.fori_loop(0, H, body, 0)
        s_copy(4, (CHUNKS - 2) % 3).wait()
        s_copy(5, (CHUNKS - 1) % 3).wait()
        plsc.subcore_barrier()
        pltpu.sync_copy(acc_sh.at[pl.ds(s * ZROWS, ZROWS)],
                        out_hbm.at[pl.ds(s * ZROWS, ZROWS)])

    @pl.when(c == 0)
    def _():
        run(xs_lo, out_lo)

    @pl.when(c == 1)
    def _():
        run(xs_hi, out_hi)


_BLK = 2000
_GRID = N // _BLK


def _row_spec(width):
    return pl.BlockSpec((_BLK, width), lambda i: (i, 0))


def _full_spec(r, cdim):
    return pl.BlockSpec((r, cdim), lambda i: (0, 0))


def _tc1_body(x_ref, w_ref, deg_ref, lo_ref, hi_ref):
    dinv = lax.rsqrt(deg_ref[...] + 1.0)
    xw = jnp.dot(x_ref[...], w_ref[...], preferred_element_type=jnp.float32)
    xs = xw * dinv
    lo_ref[...] = xs[:, :HALF]
    hi_ref[...] = xs[:, HALF:]


_tc1 = pl.pallas_call(
    _tc1_body,
    grid=(_GRID,),
    in_specs=[_row_spec(D), _full_spec(D, D), _row_spec(1)],
    out_specs=[_row_spec(HALF), _row_spec(HALF)],
    out_shape=[
        jax.ShapeDtypeStruct((N, HALF), jnp.float32),
        jax.ShapeDtypeStruct((N, HALF), jnp.float32),
    ],
)


def _tc_mid_body(alo_ref, ahi_ref, xlo_ref, xhi_ref, deg_ref, w_ref, b_ref,
                 lo_ref, hi_ref):
    dinv = lax.rsqrt(deg_ref[...] + 1.0)
    t = jnp.concatenate(
        [alo_ref[...] + xlo_ref[...], ahi_ref[...] + xhi_ref[...]], axis=1)
    h = jnp.maximum(t * dinv + b_ref[...], 0.0)
    xw = jnp.dot(h, w_ref[...], preferred_element_type=jnp.float32)
    xs = xw * dinv
    lo_ref[...] = xs[:, :HALF]
    hi_ref[...] = xs[:, HALF:]


_tc_mid = pl.pallas_call(
    _tc_mid_body,
    grid=(_GRID,),
    in_specs=[_row_spec(HALF), _row_spec(HALF), _row_spec(HALF),
              _row_spec(HALF), _row_spec(1), _full_spec(D, D),
              _full_spec(1, D)],
    out_specs=[_row_spec(HALF), _row_spec(HALF)],
    out_shape=[
        jax.ShapeDtypeStruct((N, HALF), jnp.float32),
        jax.ShapeDtypeStruct((N, HALF), jnp.float32),
    ],
)


def _tc_fin_body(alo_ref, ahi_ref, xlo_ref, xhi_ref, deg_ref, b_ref, wc_ref,
                 bc_ref, out_ref):
    dinv = lax.rsqrt(deg_ref[...] + 1.0)
    t = jnp.concatenate(
        [alo_ref[...] + xlo_ref[...], ahi_ref[...] + xhi_ref[...]], axis=1)
    h = jnp.maximum(t * dinv + b_ref[...], 0.0)
    out_ref[...] = jnp.dot(
        h, wc_ref[...], preferred_element_type=jnp.float32) + bc_ref[...]


_tc_fin = pl.pallas_call(
    _tc_fin_body,
    grid=(_GRID,),
    in_specs=[_row_spec(HALF), _row_spec(HALF), _row_spec(HALF),
              _row_spec(HALF), _row_spec(1), _full_spec(1, D),
              _full_spec(D, 1), _full_spec(1, 1)],
    out_specs=[_row_spec(1)],
    out_shape=[jax.ShapeDtypeStruct((N, 1), jnp.float32)],
)


def kernel(x, edge_index, W1, b1, W2, b2, Wc, bc):
    row = edge_index[0].astype(jnp.int32)
    col = edge_index[1].astype(jnp.int32)
    pad = E_PAD - E
    prow = (jnp.arange(pad, dtype=jnp.int32) * 79) % N
    pcol = N + jnp.arange(pad, dtype=jnp.int32) % (N_ACC - N)
    row_t = jnp.concatenate([row, prow]).reshape(NS, CHUNKS, CHUNK)
    col_t = jnp.concatenate([col, pcol]).reshape(NS, CHUNKS, CHUNK)
    rc_t = jnp.stack([row_t, col_t], axis=2)
    zdeg = jnp.zeros((N_ACC,), jnp.float32)
    zacc = jnp.zeros((ZROWS, HALF), jnp.float32)

    deg = _deg_kernel(rc_t, zdeg)
    deg2d = deg[:N].reshape(N, 1)
    b1r = b1.reshape(1, D)
    b2r = b2.reshape(1, D)
    bcr = bc.reshape(1, 1)

    xs_lo, xs_hi = _tc1(x, W1, deg2d)
    acc_lo, acc_hi = _mp_kernel(rc_t, xs_lo, xs_hi, zacc)
    xs_lo, xs_hi = _tc_mid(acc_lo, acc_hi, xs_lo, xs_hi, deg2d, W2, b1r)
    acc_lo, acc_hi = _mp_kernel(rc_t, xs_lo, xs_hi, zacc)
    xs_lo, xs_hi = _tc_mid(acc_lo, acc_hi, xs_lo, xs_hi, deg2d, W2, b2r)
    acc_lo, acc_hi = _mp_kernel(rc_t, xs_lo, xs_hi, zacc)
    (out,) = _tc_fin(acc_lo, acc_hi, xs_lo, xs_hi, deg2d, b2r, Wc, bcr)
    return out


# speedup vs baseline: 1.0487x; 1.0487x over previous
import functools

import jax
import jax.numpy as jnp
from jax import lax
from jax.experimental import pallas as pl
from jax.experimental.pallas import tpu as pltpu
from jax.experimental.pallas import tpu_sc as plsc

N = 10000
D = 256
HALF = 128
E = 160000

NC = 2
NS = 16
CHUNK = 128
CHUNKS = 80
EPT = CHUNKS * CHUNK
E_PAD = NS * EPT
N_ACC = 10112
ZROWS = N_ACC // NS

_mesh = plsc.VectorSubcoreMesh(core_axis_name="c", subcore_axis_name="s")


@functools.partial(
    pl.kernel,
    out_type=jax.ShapeDtypeStruct((2, N_ACC), jnp.float32),
    mesh=_mesh,
    scratch_types=[
        pltpu.VMEM((CHUNKS // 2, 2, CHUNK), jnp.int32),
        pltpu.VMEM((CHUNK,), jnp.float32),
        pltpu.VMEM_SHARED((N_ACC,), jnp.float32),
    ],
)
def _deg_kernel(rc_hbm, zdeg_hbm, deg_out, col_v, ones_v, deg_sh):
    c = lax.axis_index("c")
    s = lax.axis_index("s")
    hc = CHUNKS // 2
    pltpu.sync_copy(rc_hbm.at[s, pl.ds(c * hc, hc)], col_v)
    for k in range(CHUNK // 16):
        ones_v[pl.ds(k * 16, 16)] = jnp.ones((16,), jnp.float32)

    @pl.when(s == 0)
    def _():
        pltpu.sync_copy(zdeg_hbm, deg_sh)

    plsc.subcore_barrier()

    def body(j, carry):
        pltpu.sync_copy(ones_v, deg_sh.at[col_v.at[j, 1]], add=True)
        return carry

    lax.fori_loop(0, hc, body, 0)
    plsc.subcore_barrier()

    @pl.when(s == 0)
    def _():
        pltpu.sync_copy(deg_sh, deg_out.at[c])


@functools.partial(
    pl.kernel,
    out_type=(
        jax.ShapeDtypeStruct((N_ACC, HALF), jnp.float32),
        jax.ShapeDtypeStruct((N_ACC, HALF), jnp.float32),
    ),
    mesh=_mesh,
    scratch_types=[
        pltpu.VMEM((4, 2, CHUNK), jnp.int32),
        pltpu.VMEM((2, CHUNK, HALF), jnp.float32),
        pltpu.VMEM_SHARED((N_ACC, HALF), jnp.float32),
        [pltpu.SemaphoreType.DMA] * 4,
        [pltpu.SemaphoreType.DMA] * 2,
        [pltpu.SemaphoreType.DMA] * 2,
    ],
)
def _mp_kernel(rc_hbm, xs_lo, xs_hi, zacc_hbm, out_lo, out_hi,
               rc_v, buf_v, acc_sh, sem_i, sem_g, sem_s):
    c = lax.axis_index("c")
    s = lax.axis_index("s")

    def run(xs_hbm, out_hbm):
        def i_copy(j, js):
            return pltpu.make_async_copy(
                rc_hbm.at[s, j], rc_v.at[js], sem_i[js])

        def g_copy(js, db):
            return pltpu.make_async_copy(
                xs_hbm.at[rc_v.at[js, 0]], buf_v.at[db], sem_g[db])

        def s_copy(js, db):
            return pltpu.make_async_copy(
                buf_v.at[db], acc_sh.at[rc_v.at[js, 1]], sem_s[db])

        for k in range(4):
            i_copy(k, k).start()
        i_copy(0, 0).wait()
        g_copy(0, 0).start()
        pltpu.sync_copy(zacc_hbm, acc_sh.at[pl.ds(s * ZROWS, ZROWS)])
        plsc.subcore_barrier()

        H = CHUNKS // 4

        def body(h, carry):
            j0 = 4 * h
            for k in range(4):
                j = j0 + k
                js = k
                ps = (k + 3) % 4
                db = k % 2
                pb = 1 - db
                g_copy(js, db).wait()
                s_copy(js, db).start(add=True)

                if k == 0:
                    @pl.when(h > 0)
                    def _():
                        s_copy(ps, pb).wait()
                        i_copy(j + 3, ps).start()
                    i_copy(j + 1, (k + 1) % 4).wait()
                    g_copy((k + 1) % 4, pb).start()
                elif k < 3:
                    s_copy(ps, pb).wait()

                    @pl.when(h < H - 1)
                    def _():
                        i_copy(j + 3, ps).start()
                    i_copy(j + 1, (k + 1) % 4).wait()
                    g_copy((k + 1) % 4, pb).start()
                else:
                    s_copy(ps, pb).wait()

                    @pl.when(h < H - 1)
                    def _():
                        i_copy(j + 3, ps).start()
                        i_copy(j + 1, (k + 1) % 4).wait()
                        g_copy((k + 1) % 4, pb).start()

            return carry

        lax.fori_loop(0, H, body, 0)
        s_copy(3, 1).wait()
        plsc.subcore_barrier()
        pltpu.sync_copy(acc_sh.at[pl.ds(s * ZROWS, ZROWS)],
                        out_hbm.at[pl.ds(s * ZROWS, ZROWS)])

    @pl.when(c == 0)
    def _():
        run(xs_lo, out_lo)

    @pl.when(c == 1)
    def _():
        run(xs_hi, out_hi)


_BLK = 2000
_GRID = N // _BLK


def _row_spec(width):
    return pl.BlockSpec((_BLK, width), lambda i: (i, 0))


def _full_spec(r, cdim):
    return pl.BlockSpec((r, cdim), lambda i: (0, 0))


def _tc1_body(x_ref, w_ref, d0_ref, d1_ref, lo_ref, hi_ref):
    dinv = lax.rsqrt(d0_ref[...] + d1_ref[...] + 1.0)
    xw = jnp.dot(x_ref[...], w_ref[...], preferred_element_type=jnp.float32)
    xs = xw * dinv
    lo_ref[...] = xs[:, :HALF]
    hi_ref[...] = xs[:, HALF:]


_tc1 = pl.pallas_call(
    _tc1_body,
    grid=(_GRID,),
    in_specs=[_row_spec(D), _full_spec(D, D), _row_spec(1), _row_spec(1)],
    out_specs=[_row_spec(HALF), _row_spec(HALF)],
    out_shape=[
        jax.ShapeDtypeStruct((N, HALF), jnp.float32),
        jax.ShapeDtypeStruct((N, HALF), jnp.float32),
    ],
)


def _tc_mid_body(alo_ref, ahi_ref, xlo_ref, xhi_ref, d0_ref, d1_ref, w_ref,
                 b_ref, lo_ref, hi_ref):
    dinv = lax.rsqrt(d0_ref[...] + d1_ref[...] + 1.0)
    t = jnp.concatenate(
        [alo_ref[...] + xlo_ref[...], ahi_ref[...] + xhi_ref[...]], axis=1)
    h = jnp.maximum(t * dinv + b_ref[...], 0.0)
    xw = jnp.dot(h, w_ref[...], preferred_element_type=jnp.float32)
    xs = xw * dinv
    lo_ref[...] = xs[:, :HALF]
    hi_ref[...] = xs[:, HALF:]


_tc_mid = pl.pallas_call(
    _tc_mid_body,
    grid=(_GRID,),
    in_specs=[_row_spec(HALF), _row_spec(HALF), _row_spec(HALF),
              _row_spec(HALF), _row_spec(1), _row_spec(1),
              _full_spec(D, D), _full_spec(1, D)],
    out_specs=[_row_spec(HALF), _row_spec(HALF)],
    out_shape=[
        jax.ShapeDtypeStruct((N, HALF), jnp.float32),
        jax.ShapeDtypeStruct((N, HALF), jnp.float32),
    ],
)


def _tc_fin_body(alo_ref, ahi_ref, xlo_ref, xhi_ref, d0_ref, d1_ref, b_ref,
                 wc_ref, bc_ref, out_ref):
    dinv = lax.rsqrt(d0_ref[...] + d1_ref[...] + 1.0)
    t = jnp.concatenate(
        [alo_ref[...] + xlo_ref[...], ahi_ref[...] + xhi_ref[...]], axis=1)
    h = jnp.maximum(t * dinv + b_ref[...], 0.0)
    out_ref[...] = jnp.dot(
        h, wc_ref[...], preferred_element_type=jnp.float32) + bc_ref[...]


_tc_fin = pl.pallas_call(
    _tc_fin_body,
    grid=(_GRID,),
    in_specs=[_row_spec(HALF), _row_spec(HALF), _row_spec(HALF),
              _row_spec(HALF), _row_spec(1), _row_spec(1),
              _full_spec(1, D), _full_spec(D, 1), _full_spec(1, 1)],
    out_specs=[_row_spec(1)],
    out_shape=[jax.ShapeDtypeStruct((N, 1), jnp.float32)],
)


def kernel(x, edge_index, W1, b1, W2, b2, Wc, bc):
    row = edge_index[0].astype(jnp.int32)
    col = edge_index[1].astype(jnp.int32)
    pad = E_PAD - E
    prow = (jnp.arange(pad, dtype=jnp.int32) * 79) % N
    pcol = N + jnp.arange(pad, dtype=jnp.int32) % (N_ACC - N)
    row_t = jnp.concatenate([row, prow]).reshape(NS, CHUNKS, CHUNK)
    col_t = jnp.concatenate([col, pcol]).reshape(NS, CHUNKS, CHUNK)
    rc_t = jnp.stack([row_t, col_t], axis=2)
    zdeg = jnp.zeros((N_ACC,), jnp.float32)
    zacc = jnp.zeros((ZROWS, HALF), jnp.float32)

    degp = _deg_kernel(rc_t, zdeg)
    d0 = degp[0, :N].reshape(N, 1)
    d1 = degp[1, :N].reshape(N, 1)
    b1r = b1.reshape(1, D)
    b2r = b2.reshape(1, D)
    bcr = bc.reshape(1, 1)

    xs_lo, xs_hi = _tc1(x, W1, d0, d1)
    acc_lo, acc_hi = _mp_kernel(rc_t, xs_lo, xs_hi, zacc)
    xs_lo, xs_hi = _tc_mid(acc_lo, acc_hi, xs_lo, xs_hi, d0, d1, W2, b1r)
    acc_lo, acc_hi = _mp_kernel(rc_t, xs_lo, xs_hi, zacc)
    xs_lo, xs_hi = _tc_mid(acc_lo, acc_hi, xs_lo, xs_hi, d0, d1, W2, b2r)
    acc_lo, acc_hi = _mp_kernel(rc_t, xs_lo, xs_hi, zacc)
    (out,) = _tc_fin(acc_lo, acc_hi, xs_lo, xs_hi, d0, d1, b2r, Wc, bcr)
    return out
